# Initial kernel scaffold; baseline (speedup 1.0000x reference)
import jax
import jax.numpy as jnp
from jax.experimental import pallas as pl


def kernel(images):
    raise NotImplementedError("write your pallas kernel here")


# speedup vs baseline: 592.2633x; 592.2633x over previous
import functools
import jax
import jax.numpy as jnp
from jax import lax
from jax.experimental import pallas as pl
from jax.experimental.pallas import tpu as pltpu
from jax.experimental.pallas import tpu_sc as plsc

B = 4
N = 2
H = W = 384
P = H * W
TOT = B * N * P
NUM_BINS = 16
KR = 2
EPS = 1e-8

NC, NS, L = 2, 16, 16
NW = NC * NS
SPB = P // (NW // B)
NVEC = SPB // L

_mesh = plsc.VectorSubcoreMesh(core_axis_name="c", subcore_axis_name="s")


def _bsp(d):
    ad = jnp.abs(d)
    c1 = (0.5 * ad - 1.0) * (ad * ad) + (2.0 / 3.0)
    t = jnp.maximum(2.0 - ad, 0.0)
    c2 = t * t * t * (1.0 / 6.0)
    return jnp.where(ad < 1.0, c1, c2)


def _pixel(a0, a1):
    mn = jnp.minimum(a0, a1)
    mx = jnp.maximum(a0, a1)
    bw = (mx - mn) * (1.0 / NUM_BINS)
    pmin = mn - KR * bw
    inv = 1.0 / jnp.maximum(bw, EPS)
    p0 = (a0 - pmin) * inv
    p1 = (a1 - pmin) * inv
    g0 = p0.astype(jnp.int32).astype(jnp.float32)
    g1 = p1.astype(jnp.int32).astype(jnp.float32)
    i0 = jnp.clip(g0, float(KR), float(KR + NUM_BINS - 1))
    i1 = jnp.clip(g1, float(KR), float(KR + NUM_BINS - 1))
    d0 = p0 - i0
    d1 = p1 - i1
    hsum = (_bsp(d0 + 1.0) + _bsp(d0) + _bsp(d0 - 1.0) + _bsp(d0 - 2.0)
            + _bsp(d1 + 1.0) + _bsp(d1) + _bsp(d1 - 1.0) + _bsp(d1 - 2.0))
    hsum = jnp.maximum(hsum, EPS)

    def inwin(g, i):
        return (g >= i - 1.0) & (g <= i + 2.0)

    zero = jnp.zeros_like(a0)
    n0 = (jnp.where(inwin(g0, i0), _bsp(p0 - g0), zero)
          + jnp.where(inwin(g0, i1), _bsp(p1 - g0), zero))
    n1 = (jnp.where(inwin(g1, i0), _bsp(p0 - g1), zero)
          + jnp.where(inwin(g1, i1), _bsp(p1 - g1), zero))
    return n0 / hsum, n1 / hsum


@functools.partial(
    pl.kernel,
    mesh=_mesh,
    out_type=jax.ShapeDtypeStruct((TOT,), jnp.float32),
    scratch_types=[
        pltpu.VMEM((SPB,), jnp.float32),
        pltpu.VMEM((SPB,), jnp.float32),
        pltpu.VMEM((SPB,), jnp.float32),
        pltpu.VMEM((SPB,), jnp.float32),
    ],
)
def _sc_kernel(img_hbm, out_hbm, v0, v1, o0, o1):
    wid = lax.axis_index("c") * NS + lax.axis_index("s")
    b = wid // (NW // B)
    s = wid % (NW // B)
    off0 = b * (N * P) + s * SPB
    off1 = off0 + P
    pltpu.sync_copy(img_hbm.at[pl.ds(off0, SPB)], v0)
    pltpu.sync_copy(img_hbm.at[pl.ds(off1, SPB)], v1)

    def body(i, carry):
        sl = pl.ds(i * L, L)
        r0, r1 = _pixel(v0[sl], v1[sl])
        o0[sl] = r0
        o1[sl] = r1
        return carry

    lax.fori_loop(0, NVEC, body, 0)
    pltpu.sync_copy(o0, out_hbm.at[pl.ds(off0, SPB)])
    pltpu.sync_copy(o1, out_hbm.at[pl.ds(off1, SPB)])


def kernel(images):
    flat = images.reshape(TOT)
    out = _sc_kernel(flat)
    return out.reshape(B, N, H, W)


# speedup vs baseline: 786.8186x; 1.3285x over previous
import functools
import jax
import jax.numpy as jnp
from jax import lax
from jax.experimental import pallas as pl
from jax.experimental.pallas import tpu as pltpu
from jax.experimental.pallas import tpu_sc as plsc

B = 4
N = 2
H = W = 384
P = H * W
TOT = B * N * P
NUM_BINS = 16
KR = 2
EPS = 1e-8

NC, NS, L = 2, 16, 16
NW = NC * NS
SPB = P // (NW // B)
NVEC = SPB // L


def _bsp(d):
    ad = jnp.abs(d)
    c1 = (0.5 * ad - 1.0) * (ad * ad) + (2.0 / 3.0)
    t = jnp.maximum(2.0 - ad, 0.0)
    c2 = t * t * t * (1.0 / 6.0)
    return jnp.where(ad < 1.0, c1, c2)


def _pixel_fast(a0, a1, bw):
    mn = jnp.minimum(a0, a1)
    pmin = mn - KR * bw
    inv = 1.0 / jnp.maximum(bw, EPS)
    p0 = (a0 - pmin) * inv
    p1 = (a1 - pmin) * inv
    g0 = p0.astype(jnp.int32).astype(jnp.float32)
    g1 = p1.astype(jnp.int32).astype(jnp.float32)
    i0 = jnp.clip(g0, float(KR), float(KR + NUM_BINS - 1))
    i1 = jnp.clip(g1, float(KR), float(KR + NUM_BINS - 1))
    f0 = p0 - g0
    f1 = p1 - g1
    zero = jnp.zeros_like(a0)
    w0 = (0.5 * f0 - 1.0) * (f0 * f0) + (2.0 / 3.0)
    w1 = (0.5 * f1 - 1.0) * (f1 * f1) + (2.0 / 3.0)
    w0 = jnp.where(g0 >= 1.0, w0, zero)
    w1 = jnp.where(g1 >= 1.0, w1, zero)

    def wsum(p, i):
        ad = p - i + 2.0
        c1 = (0.5 * ad - 1.0) * (ad * ad) + (2.0 / 3.0)
        t = jnp.maximum(2.0 - ad, 0.0)
        b2 = jnp.where(ad < 1.0, c1, t * t * t * (1.0 / 6.0))
        q = jnp.maximum(-1.0 - (p - i), 0.0)
        return 1.0 - b2 - q * q * q * (1.0 / 6.0)

    hsum = jnp.maximum(wsum(p0, i0) + wsum(p1, i1), EPS)
    c01 = jnp.where((g0 >= i1 - 1.0) & (g0 <= i1 + 2.0), _bsp(p1 - g0), zero)
    c10 = jnp.where((g1 >= i0 - 1.0) & (g1 <= i0 + 2.0), _bsp(p0 - g1), zero)
    rec = 1.0 / hsum
    return (w0 + c01) * rec, (w1 + c10) * rec


@functools.lru_cache(maxsize=1)
def _build():
    mesh = plsc.VectorSubcoreMesh(core_axis_name="c", subcore_axis_name="s")

    @functools.partial(
        pl.kernel,
        mesh=mesh,
        out_type=jax.ShapeDtypeStruct((TOT,), jnp.float32),
        scratch_types=[
            pltpu.VMEM((SPB,), jnp.float32),
            pltpu.VMEM((SPB,), jnp.float32),
            pltpu.VMEM((SPB,), jnp.float32),
            pltpu.VMEM((SPB,), jnp.float32),
        ],
    )
    def _sc_kernel(img_hbm, out_hbm, v0, v1, o0, o1):
        wid = lax.axis_index("c") * NS + lax.axis_index("s")
        b = wid // (NW // B)
        s = wid % (NW // B)
        off0 = b * (N * P) + s * SPB
        off1 = off0 + P
        pltpu.sync_copy(img_hbm.at[pl.ds(off0, SPB)], v0)
        pltpu.sync_copy(img_hbm.at[pl.ds(off1, SPB)], v1)

        def body(i, carry):
            sl = pl.ds(i * L, L)
            a0 = v0[sl]
            a1 = v1[sl]
            bw = (jnp.maximum(a0, a1) - jnp.minimum(a0, a1)) * (1.0 / NUM_BINS)
            r0, r1 = _pixel_fast(a0, a1, bw)
            o0[sl] = r0
            o1[sl] = r1

            return carry

        lax.fori_loop(0, NVEC, body, 0)
        pltpu.sync_copy(o0, out_hbm.at[pl.ds(off0, SPB)])
        pltpu.sync_copy(o1, out_hbm.at[pl.ds(off1, SPB)])

    return _sc_kernel


def kernel(images):
    flat = images.reshape(TOT)
    out = _build()(flat)
    return out.reshape(B, N, H, W)
